# baseline (speedup 1.0000x reference)
import functools

import jax
import jax.numpy as jnp
from jax import lax
from jax.experimental import pallas as pl
from jax.experimental.pallas import tpu as pltpu
from jax.experimental.pallas import tpu_sc as plsc

B, S, F = 4096, 200, 27
N_CAT, E = 1000, 32
OUT_F = E + (F - 1)
L = 16
NC, NS = 2, 16
NW = NC * NS
RU = 8
UNITS = S // RU
GPR = B // L
GU = 8
NIT = RU * GPR // GU
NREM = (F - 1) * UNITS
PAIRS = 23
EXP_BIAS = 0x4B000000


def _build_sc_call():
    mesh = plsc.VectorSubcoreMesh(core_axis_name="c", subcore_axis_name="s")

    @functools.partial(
        pl.kernel,
        mesh=mesh,
        compiler_params=pltpu.CompilerParams(needs_layout_passes=False),
        out_type=jax.ShapeDtypeStruct((OUT_F, S, B), jnp.float32),
        scratch_types=[
            pltpu.VMEM((E * N_CAT,), jnp.float32),
            pltpu.VMEM((RU, B), jnp.float32),
            pltpu.VMEM((RU, B), jnp.float32),
            pltpu.SemaphoreType.DMA,
            pltpu.SemaphoreType.DMA,
            pltpu.SemaphoreType.DMA,
            pltpu.SemaphoreType.DMA,
            pltpu.SemaphoreType.DMA,
        ],
    )
    def sc_fn(in_hbm, tab_hbm, out_hbm, tab_v, xe0, xe1,
              sem_t, sem_x0, sem_x1, sem_o0, sem_o1):
        wid = lax.axis_index("s") * NC + lax.axis_index("c")
        dk = wid * N_CAT - EXP_BIAS
        rem0 = wid * NREM // NW
        nrem = (wid + 1) * NREM // NW - rem0

        def x_slice(u):
            return in_hbm.at[0, pl.ds(pl.multiple_of(u * RU, RU), RU), :]

        def out_slice(u):
            return out_hbm.at[wid, pl.ds(pl.multiple_of(u * RU, RU), RU), :]

        def rem_slices(j):
            k = rem0 + j
            p = 1 + k // UNITS
            s0 = pl.multiple_of((k % UNITS) * RU, RU)
            return (in_hbm.at[p, pl.ds(s0, RU), :],
                    out_hbm.at[p + E - 1, pl.ds(s0, RU), :])

        def emb_u0(p):
            return jnp.where(p < 21, p, 2 * p - 21)

        def active1(p):
            return (p < nrem) | (p >= 21)

        def load0(p):
            pltpu.async_copy(x_slice(emb_u0(p)), xe0, sem_x0)

        def load1(p):
            @pl.when(p < nrem)
            def _():
                src, _dst = rem_slices(p)
                pltpu.async_copy(src, xe1, sem_x1)

            @pl.when(p >= 21)
            def _():
                pltpu.async_copy(x_slice(2 * p - 20), xe1, sem_x1)

        def store1(p):
            @pl.when(p < nrem)
            def _():
                _src, dst = rem_slices(p)
                pltpu.async_copy(xe1, dst, sem_o1)

            @pl.when(p >= 21)
            def _():
                pltpu.async_copy(xe1, out_slice(2 * p - 20), sem_o1)

        def wait_load(xe, sem):
            pltpu.make_async_copy(x_slice(0), xe, sem).wait()

        def drain_store(xe, sem):
            pltpu.make_async_copy(xe, out_slice(0), sem).wait()

        pltpu.async_copy(tab_hbm, tab_v, sem_t)
        load0(jnp.int32(0))
        load1(jnp.int32(0))
        pltpu.make_async_copy(tab_hbm, tab_v, sem_t).wait()

        def compute(xe):
            def gbody(gi, carry):
                r = gi // (GPR // GU)
                lbase = (gi % (GPR // GU)) * (GU * L)
                for k in range(GU):
                    l0 = pl.multiple_of(lbase + k * L, L)
                    x = xe[r, pl.ds(l0, L)]
                    t = x * 255.0 + 8388608.0
                    idx = plsc.bitcast(t, jnp.int32) + dk
                    xe[r, pl.ds(l0, L)] = plsc.load_gather(tab_v, [idx])
                return carry
            lax.fori_loop(0, NIT, gbody, 0)

        def step(p, carry):
            wait_load(xe0, sem_x0)
            compute(xe0)
            pltpu.async_copy(xe0, out_slice(emb_u0(p)), sem_o0)
            @pl.when(active1(p))
            def _(p=p):
                wait_load(xe1, sem_x1)

            @pl.when(p >= 21)
            def _(p=p):
                compute(xe1)
            store1(p)

            @pl.when(p < PAIRS - 1)
            def _prefetch(p=p):
                drain_store(xe0, sem_o0)
                load0(p + 1)

                @pl.when(active1(p))
                def _(p=p):
                    drain_store(xe1, sem_o1)

                @pl.when(active1(p + 1))
                def _(p=p):
                    load1(p + 1)
            return carry

        lax.fori_loop(0, PAIRS, step, 0)
        drain_store(xe0, sem_o0)
        drain_store(xe1, sem_o1)

    return sc_fn


_sc_call = _build_sc_call()


def kernel(inputs, table):
    in_pm = inputs.transpose(2, 1, 0)
    tab_dm = table.transpose(1, 0).reshape(E * N_CAT)
    out_pm = _sc_call(in_pm, tab_dm)
    return out_pm.transpose(2, 1, 0)


# speedup vs baseline: 1.5848x; 1.5848x over previous
import functools

import jax
import jax.numpy as jnp
from jax import lax
from jax.experimental import pallas as pl
from jax.experimental.pallas import tpu as pltpu
from jax.experimental.pallas import tpu_sc as plsc

B, S, F = 4096, 200, 27
N_CAT, E = 1000, 32
OUT_F = E + (F - 1)
L = 16
NC, NS = 2, 16
NW = NC * NS
RU = 8
UNITS = S // RU
GPR = B // L
GU = 8
NIT = RU * GPR // GU
NREM = (F - 1) * UNITS
PAIRS = 23
EXP_BIAS = 0x4B000000


def _build_sc_call():
    mesh = plsc.VectorSubcoreMesh(core_axis_name="c", subcore_axis_name="s")

    @functools.partial(
        pl.kernel,
        mesh=mesh,
        compiler_params=pltpu.CompilerParams(needs_layout_passes=False),
        out_type=jax.ShapeDtypeStruct((OUT_F, S, B), jnp.float32),
        scratch_types=[
            pltpu.VMEM((E * N_CAT,), jnp.float32),
            pltpu.VMEM((RU, B), jnp.float32),
            pltpu.VMEM((RU, B), jnp.float32),
            pltpu.SemaphoreType.DMA,
            pltpu.SemaphoreType.DMA,
            pltpu.SemaphoreType.DMA,
            pltpu.SemaphoreType.DMA,
            pltpu.SemaphoreType.DMA,
        ],
    )
    def sc_fn(in_hbm, tab_hbm, out_hbm, tab_v, xe0, xe1,
              sem_t, sem_x0, sem_x1, sem_o0, sem_o1):
        wid = lax.axis_index("s") * NC + lax.axis_index("c")
        dk = wid * N_CAT - EXP_BIAS
        rem0 = wid * NREM // NW
        nrem = (wid + 1) * NREM // NW - rem0

        def x_slice(u):
            return in_hbm.at[0, pl.ds(pl.multiple_of(u * RU, RU), RU), :]

        def out_slice(u):
            return out_hbm.at[wid, pl.ds(pl.multiple_of(u * RU, RU), RU), :]

        def rem_slices(j):
            k = rem0 + j
            p = 1 + k // UNITS
            s0 = pl.multiple_of((k % UNITS) * RU, RU)
            return (in_hbm.at[p, pl.ds(s0, RU), :],
                    out_hbm.at[p + E - 1, pl.ds(s0, RU), :])

        def emb_u0(p):
            return jnp.where(p < 21, p, 2 * p - 21)

        def active1(p):
            return (p < nrem) | (p >= 21)

        def load0(p):
            pltpu.async_copy(x_slice(emb_u0(p)), xe0, sem_x0)

        def load1(p):
            @pl.when(p < nrem)
            def _():
                src, _dst = rem_slices(p)
                pltpu.async_copy(src, xe1, sem_x1)

            @pl.when(p >= 21)
            def _():
                pltpu.async_copy(x_slice(2 * p - 20), xe1, sem_x1)

        def store1(p):
            @pl.when(p < nrem)
            def _():
                _src, dst = rem_slices(p)
                pltpu.async_copy(xe1, dst, sem_o1)

            @pl.when(p >= 21)
            def _():
                pltpu.async_copy(xe1, out_slice(2 * p - 20), sem_o1)

        def wait_load(xe, sem):
            pltpu.make_async_copy(x_slice(0), xe, sem).wait()

        def drain_store(xe, sem):
            pltpu.make_async_copy(xe, out_slice(0), sem).wait()

        pltpu.async_copy(tab_hbm, tab_v, sem_t)
        load0(jnp.int32(0))
        load1(jnp.int32(0))
        pltpu.make_async_copy(tab_hbm, tab_v, sem_t).wait()

        def compute(xe):
            @plsc.parallel_loop(0, RU * GPR, 1, unroll=GU)
            def _body(g):
                r = g // GPR
                l0 = pl.multiple_of((g % GPR) * L, L)
                x = xe[r, pl.ds(l0, L)]
                t = x * 255.0 + 8388608.0
                idx = plsc.bitcast(t, jnp.int32) + dk
                xe[r, pl.ds(l0, L)] = plsc.load_gather(tab_v, [idx])

        def step(p, carry):
            wait_load(xe0, sem_x0)
            compute(xe0)
            pltpu.async_copy(xe0, out_slice(emb_u0(p)), sem_o0)
            @pl.when(active1(p))
            def _(p=p):
                wait_load(xe1, sem_x1)

            @pl.when(p >= 21)
            def _(p=p):
                compute(xe1)
            store1(p)

            @pl.when(p < PAIRS - 1)
            def _prefetch(p=p):
                drain_store(xe0, sem_o0)
                load0(p + 1)

                @pl.when(active1(p))
                def _(p=p):
                    drain_store(xe1, sem_o1)

                @pl.when(active1(p + 1))
                def _(p=p):
                    load1(p + 1)
            return carry

        lax.fori_loop(0, PAIRS, step, 0)
        drain_store(xe0, sem_o0)
        drain_store(xe1, sem_o1)

    return sc_fn


_sc_call = _build_sc_call()


def kernel(inputs, table):
    in_pm = inputs.transpose(2, 1, 0)
    tab_dm = table.transpose(1, 0).reshape(E * N_CAT)
    out_pm = _sc_call(in_pm, tab_dm)
    return out_pm.transpose(2, 1, 0)


# speedup vs baseline: 1.8356x; 1.1582x over previous
import functools

import jax
import jax.numpy as jnp
from jax import lax
from jax.experimental import pallas as pl
from jax.experimental.pallas import tpu as pltpu
from jax.experimental.pallas import tpu_sc as plsc

B, S, F = 4096, 200, 27
N_CAT, E = 1000, 32
OUT_F = E + (F - 1)
L = 16
NC, NS = 2, 16
NW = NC * NS
RU = 8
HB = B // 2
UNITS = S // RU
GPB = RU * HB // L
GU = 8
NREM = (F - 1) * UNITS * 2
PAIRS = 33
EXP_BIAS = 0x4B000000


def _build_sc_call():
    mesh = plsc.VectorSubcoreMesh(core_axis_name="c", subcore_axis_name="s")

    @functools.partial(
        pl.kernel,
        mesh=mesh,
        compiler_params=pltpu.CompilerParams(needs_layout_passes=False),
        out_type=jax.ShapeDtypeStruct((OUT_F, S, B), jnp.float32),
        scratch_types=[
            pltpu.VMEM((E * N_CAT,), jnp.float32),
            pltpu.VMEM((RU, HB), jnp.float32),
            pltpu.VMEM((RU, HB), jnp.float32),
            pltpu.VMEM((RU, HB), jnp.float32),
            pltpu.VMEM((RU, HB), jnp.float32),
            pltpu.SemaphoreType.DMA,
            pltpu.SemaphoreType.DMA,
            pltpu.SemaphoreType.DMA,
            pltpu.SemaphoreType.DMA,
            pltpu.SemaphoreType.DMA,
        ],
    )
    def sc_fn(in_hbm, tab_hbm, out_hbm, tab_v, xe0, xe1, e20, e21,
              sem_t, sem_x0, sem_x1, sem_o0, sem_o1):
        wid = lax.axis_index("s") * NC + lax.axis_index("c")
        a2 = (wid // 2) * 2
        lh = (wid % 2) * HB
        dk0 = a2 * N_CAT - EXP_BIAS
        dk1 = dk0 + N_CAT
        rem0 = wid * NREM // NW
        nrem = (wid + 1) * NREM // NW - rem0

        def x_slice(u):
            return in_hbm.at[0, pl.ds(pl.multiple_of(u * RU, RU), RU),
                             pl.ds(pl.multiple_of(lh, HB), HB)]

        def out_slice(d_off, u):
            return out_hbm.at[a2 + d_off,
                              pl.ds(pl.multiple_of(u * RU, RU), RU),
                              pl.ds(pl.multiple_of(lh, HB), HB)]

        def rem_slices(j):
            k = rem0 + j
            p = 1 + k // (2 * UNITS)
            q = k % (2 * UNITS)
            s0 = pl.multiple_of((q // 2) * RU, RU)
            lo = pl.multiple_of((q % 2) * HB, HB)
            return (in_hbm.at[p, pl.ds(s0, RU), pl.ds(lo, HB)],
                    out_hbm.at[p + E - 1, pl.ds(s0, RU), pl.ds(lo, HB)])

        def j1(p):
            return jnp.where(p < UNITS, p, 2 * p - 24)

        def active1(p):
            return (p < UNITS) | (2 * p - 24 < nrem)

        def load0(p):
            @pl.when(p < UNITS)
            def _():
                pltpu.async_copy(x_slice(p), xe0, sem_x0)

            @pl.when(p >= UNITS)
            def _():
                src, _ = rem_slices(2 * p - 25)
                pltpu.async_copy(src, xe0, sem_x0)

        def load1(p):
            @pl.when(active1(p))
            def _():
                src, _ = rem_slices(j1(p))
                pltpu.async_copy(src, xe1, sem_x1)

        def store1(p):
            @pl.when(active1(p))
            def _():
                _, dst = rem_slices(j1(p))
                pltpu.async_copy(xe1, dst, sem_o1)

        def wait_load(xe, sem):
            pltpu.make_async_copy(x_slice(0), xe, sem).wait()

        def drain1(xe, sem):
            pltpu.make_async_copy(xe, out_slice(0, 0), sem).wait()

        pltpu.async_copy(tab_hbm, tab_v, sem_t)
        load0(jnp.int32(0))
        load1(jnp.int32(0))
        pltpu.make_async_copy(tab_hbm, tab_v, sem_t).wait()

        def compute2(xe, e2):
            @plsc.parallel_loop(0, GPB, 1, unroll=GU)
            def _body(g):
                r = g // (HB // L)
                l0 = pl.multiple_of((g % (HB // L)) * L, L)
                x = xe[r, pl.ds(l0, L)]
                t = x * 255.0 + 8388608.0
                bits = plsc.bitcast(t, jnp.int32)
                v0 = plsc.load_gather(tab_v, [bits + dk0])
                v1 = plsc.load_gather(tab_v, [bits + dk1])
                xe[r, pl.ds(l0, L)] = v0
                e2[r, pl.ds(l0, L)] = v1

        def step(p, carry):
            wait_load(xe0, sem_x0)

            @pl.when(p < UNITS)
            def _(p=p):
                compute2(xe0, e20)
                pltpu.async_copy(xe0, out_slice(0, p), sem_o0)
                pltpu.async_copy(e20, out_slice(1, p), sem_o0)

            @pl.when(p >= UNITS)
            def _(p=p):
                _, dst = rem_slices(2 * p - 25)
                pltpu.async_copy(xe0, dst, sem_o0)

            @pl.when(active1(p))
            def _(p=p):
                wait_load(xe1, sem_x1)
            store1(p)

            @pl.when(p < PAIRS - 1)
            def _prefetch(p=p):
                drain1(xe0, sem_o0)

                @pl.when(p < UNITS)
                def _(p=p):
                    drain1(e20, sem_o0)
                load0(p + 1)

                @pl.when(active1(p))
                def _(p=p):
                    drain1(xe1, sem_o1)

                @pl.when(active1(p + 1))
                def _(p=p):
                    load1(p + 1)
            return carry

        lax.fori_loop(0, PAIRS, step, 0)
        drain1(xe0, sem_o0)

        @pl.when(active1(PAIRS - 1))
        def _():
            drain1(xe1, sem_o1)

    return sc_fn


_sc_call = _build_sc_call()


def kernel(inputs, table):
    in_pm = inputs.transpose(2, 1, 0)
    tab_dm = table.transpose(1, 0).reshape(E * N_CAT)
    out_pm = _sc_call(in_pm, tab_dm)
    return out_pm.transpose(2, 1, 0)


# speedup vs baseline: 1.8947x; 1.0322x over previous
import functools

import jax
import jax.numpy as jnp
from jax import lax
from jax.experimental import pallas as pl
from jax.experimental.pallas import tpu as pltpu
from jax.experimental.pallas import tpu_sc as plsc

B, S, F = 4096, 200, 27
N_CAT, E = 1000, 32
OUT_F = E + (F - 1)
L = 16
NC, NS = 2, 16
NW = NC * NS
RU = 8
QB = B // 4
UNITS = S // RU
GPB = RU * QB // L
GU = 4
NREM = (F - 1) * UNITS * 4
PAIRS = 54
EXP_BIAS = 0x4B000000


def _build_sc_call():
    mesh = plsc.VectorSubcoreMesh(core_axis_name="c", subcore_axis_name="s")

    @functools.partial(
        pl.kernel,
        mesh=mesh,
        compiler_params=pltpu.CompilerParams(needs_layout_passes=False),
        out_type=jax.ShapeDtypeStruct((OUT_F, S, B), jnp.float32),
        scratch_types=[
            pltpu.VMEM((E * N_CAT,), jnp.float32),
            pltpu.VMEM((RU, QB), jnp.float32),
            pltpu.VMEM((RU, QB), jnp.float32),
            pltpu.VMEM((RU, QB), jnp.float32),
            pltpu.VMEM((RU, QB), jnp.float32),
            pltpu.VMEM((RU, QB), jnp.float32),
            pltpu.VMEM((RU, QB), jnp.float32),
            pltpu.VMEM((RU, QB), jnp.float32),
            pltpu.VMEM((RU, QB), jnp.float32),
            pltpu.SemaphoreType.DMA,
            pltpu.SemaphoreType.DMA,
            pltpu.SemaphoreType.DMA,
            pltpu.SemaphoreType.DMA,
            pltpu.SemaphoreType.DMA,
        ],
    )
    def sc_fn(in_hbm, tab_hbm, out_hbm, tab_v,
              xe0, xe1, e20, e21, e30, e31, e40, e41,
              sem_t, sem_x0, sem_x1, sem_o0, sem_o1):
        wid = lax.axis_index("s") * NC + lax.axis_index("c")
        a4 = (wid // 4) * 4
        lh = (wid % 4) * QB
        dks = [a4 * N_CAT - EXP_BIAS + i * N_CAT for i in range(4)]
        rem0 = wid * NREM // NW
        nrem = (wid + 1) * NREM // NW - rem0

        def x_slice(u):
            return in_hbm.at[0, pl.ds(pl.multiple_of(u * RU, RU), RU),
                             pl.ds(pl.multiple_of(lh, QB), QB)]

        def out_slice(d_off, u):
            return out_hbm.at[a4 + d_off,
                              pl.ds(pl.multiple_of(u * RU, RU), RU),
                              pl.ds(pl.multiple_of(lh, QB), QB)]

        def rem_slices(j):
            k = rem0 + j
            p = 1 + k // (4 * UNITS)
            q = k % (4 * UNITS)
            s0 = pl.multiple_of((q // 4) * RU, RU)
            lo = pl.multiple_of((q % 4) * QB, QB)
            return (in_hbm.at[p, pl.ds(s0, RU), pl.ds(lo, QB)],
                    out_hbm.at[p + E - 1, pl.ds(s0, RU), pl.ds(lo, QB)])

        def j1(p):
            return jnp.where(p < UNITS, p, 2 * p - 24)

        def active0(p):
            return (p < UNITS) | (2 * p - 25 < nrem)

        def active1(p):
            return (p < UNITS) | (2 * p - 24 < nrem)

        def load0(p):
            @pl.when(p < UNITS)
            def _():
                pltpu.async_copy(x_slice(p), xe0, sem_x0)

            @pl.when((p >= UNITS) & (2 * p - 25 < nrem))
            def _():
                src, _ = rem_slices(2 * p - 25)
                pltpu.async_copy(src, xe0, sem_x0)

        def load1(p):
            @pl.when(active1(p))
            def _():
                src, _ = rem_slices(j1(p))
                pltpu.async_copy(src, xe1, sem_x1)

        def store1(p):
            @pl.when(active1(p))
            def _():
                _, dst = rem_slices(j1(p))
                pltpu.async_copy(xe1, dst, sem_o1)

        def wait_load(xe, sem):
            pltpu.make_async_copy(x_slice(0), xe, sem).wait()

        def drain1(xe, sem):
            pltpu.make_async_copy(xe, out_slice(0, 0), sem).wait()

        pltpu.async_copy(tab_hbm, tab_v, sem_t)
        load0(jnp.int32(0))
        load1(jnp.int32(0))
        pltpu.make_async_copy(tab_hbm, tab_v, sem_t).wait()

        def compute4(xe, es):
            @plsc.parallel_loop(0, GPB, 1, unroll=GU)
            def _body(g):
                r = g // (QB // L)
                l0 = pl.multiple_of((g % (QB // L)) * L, L)
                x = xe[r, pl.ds(l0, L)]
                t = x * 255.0 + 8388608.0
                bits = plsc.bitcast(t, jnp.int32)
                vals = [plsc.load_gather(tab_v, [bits + dk]) for dk in dks]
                xe[r, pl.ds(l0, L)] = vals[0]
                for e, v in zip(es, vals[1:]):
                    e[r, pl.ds(l0, L)] = v

        def step(p, carry):
            @pl.when(active0(p))
            def _(p=p):
                wait_load(xe0, sem_x0)

            @pl.when(p < UNITS)
            def _(p=p):
                compute4(xe0, (e20, e30, e40))
                pltpu.async_copy(xe0, out_slice(0, p), sem_o0)
                pltpu.async_copy(e20, out_slice(1, p), sem_o0)
                pltpu.async_copy(e30, out_slice(2, p), sem_o0)
                pltpu.async_copy(e40, out_slice(3, p), sem_o0)

            @pl.when((p >= UNITS) & (2 * p - 25 < nrem))
            def _(p=p):
                _, dst = rem_slices(2 * p - 25)
                pltpu.async_copy(xe0, dst, sem_o0)

            @pl.when(active1(p))
            def _(p=p):
                wait_load(xe1, sem_x1)
            store1(p)

            @pl.when(p < PAIRS - 1)
            def _prefetch(p=p):
                @pl.when(active0(p))
                def _(p=p):
                    drain1(xe0, sem_o0)

                @pl.when(p < UNITS)
                def _(p=p):
                    drain1(e20, sem_o0)
                    drain1(e30, sem_o0)
                    drain1(e40, sem_o0)
                load0(p + 1)

                @pl.when(active1(p))
                def _(p=p):
                    drain1(xe1, sem_o1)

                @pl.when(active1(p + 1))
                def _(p=p):
                    load1(p + 1)
            return carry

        lax.fori_loop(0, PAIRS, step, 0)

        @pl.when(active0(PAIRS - 1))
        def _():
            drain1(xe0, sem_o0)

        @pl.when(active1(PAIRS - 1))
        def _():
            drain1(xe1, sem_o1)

    return sc_fn


_sc_call = _build_sc_call()


def kernel(inputs, table):
    in_pm = inputs.transpose(2, 1, 0)
    tab_dm = table.transpose(1, 0).reshape(E * N_CAT)
    out_pm = _sc_call(in_pm, tab_dm)
    return out_pm.transpose(2, 1, 0)


# speedup vs baseline: 1.9412x; 1.0246x over previous
import functools

import jax
import jax.numpy as jnp
from jax import lax
from jax.experimental import pallas as pl
from jax.experimental.pallas import tpu as pltpu
from jax.experimental.pallas import tpu_sc as plsc

B, S, F = 4096, 200, 27
N_CAT, E = 1000, 32
OUT_F = E + (F - 1)
L = 16
NC, NS = 2, 16
NW = NC * NS
RU = 8
QB = B // 4
UNITS = S // RU
GPB = RU * QB // L
GU = 4
NREM = (F - 1) * UNITS * 4
PAIRS = 54
EXP_BIAS = 0x4B000000


def _build_sc_call():
    mesh = plsc.VectorSubcoreMesh(core_axis_name="c", subcore_axis_name="s")

    @functools.partial(
        pl.kernel,
        mesh=mesh,
        compiler_params=pltpu.CompilerParams(needs_layout_passes=False),
        out_type=jax.ShapeDtypeStruct((OUT_F, S, B), jnp.float32),
        scratch_types=[
            pltpu.VMEM((E * N_CAT,), jnp.float32),
            pltpu.VMEM((RU, QB), jnp.float32),
            pltpu.VMEM((RU, QB), jnp.float32),
            pltpu.VMEM((RU, QB), jnp.float32),
            pltpu.VMEM((RU, QB), jnp.float32),
            pltpu.VMEM((RU, QB), jnp.float32),
            pltpu.VMEM((RU, QB), jnp.float32),
            pltpu.VMEM((RU, QB), jnp.float32),
            pltpu.VMEM((RU, QB), jnp.float32),
            pltpu.SemaphoreType.DMA,
            pltpu.SemaphoreType.DMA,
            pltpu.SemaphoreType.DMA,
            pltpu.SemaphoreType.DMA,
            pltpu.SemaphoreType.DMA,
        ],
    )
    def sc_fn(in_hbm, tab_hbm, out_hbm, tab_v,
              xe0, xe1, e20, e21, e30, e31, e40, e41,
              sem_t, sem_x0, sem_x1, sem_o0, sem_o1):
        wid = lax.axis_index("s") * NC + lax.axis_index("c")
        a4 = (wid // 4) * 4
        lh = (wid % 4) * QB
        dks = [a4 * N_CAT - EXP_BIAS + i * N_CAT for i in range(4)]
        rem0 = wid * NREM // NW
        nrem = (wid + 1) * NREM // NW - rem0

        def x_slice(u):
            return in_hbm.at[0, pl.ds(pl.multiple_of(u * RU, RU), RU),
                             pl.ds(pl.multiple_of(lh, QB), QB)]

        def out_slice(d_off, u):
            return out_hbm.at[a4 + d_off,
                              pl.ds(pl.multiple_of(u * RU, RU), RU),
                              pl.ds(pl.multiple_of(lh, QB), QB)]

        def rem_slices(j):
            k = rem0 + j
            p = 1 + k // (4 * UNITS)
            q = k % (4 * UNITS)
            s0 = pl.multiple_of((q // 4) * RU, RU)
            lo = pl.multiple_of((q % 4) * QB, QB)
            return (in_hbm.at[p, pl.ds(s0, RU), pl.ds(lo, QB)],
                    out_hbm.at[p + E - 1, pl.ds(s0, RU), pl.ds(lo, QB)])

        def j1(p):
            return jnp.where(p < UNITS, p, 2 * p - 24)

        def active0(p):
            return (p < UNITS) | (2 * p - 25 < nrem)

        def active1(p):
            return (p < UNITS) | (2 * p - 24 < nrem)

        def load0(p):
            @pl.when(p < UNITS)
            def _():
                pltpu.async_copy(x_slice(p), xe0, sem_x0)

            @pl.when((p >= UNITS) & (2 * p - 25 < nrem))
            def _():
                src, _ = rem_slices(2 * p - 25)
                pltpu.async_copy(src, xe0, sem_x0)

        def load1(p):
            @pl.when(active1(p))
            def _():
                src, _ = rem_slices(j1(p))
                pltpu.async_copy(src, xe1, sem_x1)

        def store1(p):
            @pl.when(active1(p))
            def _():
                _, dst = rem_slices(j1(p))
                pltpu.async_copy(xe1, dst, sem_o1)

        def wait_load(xe, sem):
            pltpu.make_async_copy(x_slice(0), xe, sem).wait()

        def drain1(xe, sem):
            pltpu.make_async_copy(xe, out_slice(0, 0), sem).wait()

        pltpu.async_copy(tab_hbm, tab_v, sem_t)
        load0(jnp.int32(0))
        load1(jnp.int32(0))
        pltpu.make_async_copy(tab_hbm, tab_v, sem_t).wait()

        def compute4(xe, es):
            @plsc.parallel_loop(0, GPB, 1, unroll=GU)
            def _body(g):
                r = g // (QB // L)
                l0 = pl.multiple_of((g % (QB // L)) * L, L)
                x = xe[r, pl.ds(l0, L)]
                t = x * 255.0 + 8388608.0
                bits = plsc.bitcast(t, jnp.int32)
                vals = [plsc.load_gather(tab_v, [bits + dk]) for dk in dks]
                xe[r, pl.ds(l0, L)] = vals[0]
                for e, v in zip(es, vals[1:]):
                    e[r, pl.ds(l0, L)] = v

        def step(p, carry):
            @pl.when(active1(p))
            def _(p=p):
                wait_load(xe1, sem_x1)
            store1(p)

            @pl.when(active0(p))
            def _(p=p):
                wait_load(xe0, sem_x0)

            @pl.when(p < UNITS)
            def _(p=p):
                compute4(xe0, (e20, e30, e40))
                pltpu.async_copy(xe0, out_slice(0, p), sem_o0)
                pltpu.async_copy(e20, out_slice(1, p), sem_o0)
                pltpu.async_copy(e30, out_slice(2, p), sem_o0)
                pltpu.async_copy(e40, out_slice(3, p), sem_o0)

            @pl.when((p >= UNITS) & (2 * p - 25 < nrem))
            def _(p=p):
                _, dst = rem_slices(2 * p - 25)
                pltpu.async_copy(xe0, dst, sem_o0)

            @pl.when(p < PAIRS - 1)
            def _prefetch(p=p):
                @pl.when(active0(p))
                def _(p=p):
                    drain1(xe0, sem_o0)

                @pl.when(p < UNITS)
                def _(p=p):
                    drain1(e20, sem_o0)
                    drain1(e30, sem_o0)
                    drain1(e40, sem_o0)
                load0(p + 1)

                @pl.when(active1(p))
                def _(p=p):
                    drain1(xe1, sem_o1)

                @pl.when(active1(p + 1))
                def _(p=p):
                    load1(p + 1)
            return carry

        lax.fori_loop(0, PAIRS, step, 0)

        @pl.when(active0(PAIRS - 1))
        def _():
            drain1(xe0, sem_o0)

        @pl.when(active1(PAIRS - 1))
        def _():
            drain1(xe1, sem_o1)

    return sc_fn


_sc_call = _build_sc_call()


def kernel(inputs, table):
    in_pm = inputs.transpose(2, 1, 0)
    tab_dm = table.transpose(1, 0).reshape(E * N_CAT)
    out_pm = _sc_call(in_pm, tab_dm)
    return out_pm.transpose(2, 1, 0)


# speedup vs baseline: 2.3955x; 1.2340x over previous
import functools

import jax
import jax.numpy as jnp
from jax import lax
from jax.experimental import pallas as pl
from jax.experimental.pallas import tpu as pltpu
from jax.experimental.pallas import tpu_sc as plsc

B, S, F = 4096, 200, 27
N_CAT, E = 1000, 32
OUT_F = E + (F - 1)
L = 16
NC, NS = 2, 16
NW = NC * NS
RU = 8
QB = B // 4
UNITS = S // RU
GPB = RU * QB // L
GU = 4
NREM = (F - 1) * UNITS * 4
RPB = 3
EXP_BIAS = 0x4B000000


def _build_sc_call():
    mesh = plsc.VectorSubcoreMesh(core_axis_name="c", subcore_axis_name="s")

    @functools.partial(
        pl.kernel,
        mesh=mesh,
        compiler_params=pltpu.CompilerParams(needs_layout_passes=False),
        out_type=jax.ShapeDtypeStruct((OUT_F, S, B), jnp.float32),
        scratch_types=[
            pltpu.VMEM((E * N_CAT,), jnp.float32),
            [pltpu.VMEM((RU, QB), jnp.float32)] * 2,
            [[pltpu.VMEM((RU, QB), jnp.float32)] * 4] * 2,
            [pltpu.VMEM((RU, QB), jnp.float32)] * 2,
            pltpu.SemaphoreType.DMA,
            [pltpu.SemaphoreType.DMA] * 2,
            [pltpu.SemaphoreType.DMA] * 2,
            [pltpu.SemaphoreType.DMA] * 2,
            [pltpu.SemaphoreType.DMA] * 2,
        ],
    )
    def sc_fn(in_hbm, tab_hbm, out_hbm, tab_v, xs, outs, rems,
              sem_t, sem_x, sem_o, sem_rl, sem_rs):
        wid = lax.axis_index("s") * NC + lax.axis_index("c")
        a4 = (wid // 4) * 4
        lh = (wid % 4) * QB
        dks = [a4 * N_CAT - EXP_BIAS + i * N_CAT for i in range(4)]
        rem0 = wid * NREM // NW
        nrem = (wid + 1) * NREM // NW - rem0

        def x_slice(u):
            return in_hbm.at[0, pl.ds(pl.multiple_of(u * RU, RU), RU),
                             pl.ds(pl.multiple_of(lh, QB), QB)]

        def out_slice(d_off, u):
            return out_hbm.at[a4 + d_off,
                              pl.ds(pl.multiple_of(u * RU, RU), RU),
                              pl.ds(pl.multiple_of(lh, QB), QB)]

        def rem_slices(j):
            k = rem0 + j
            p = 1 + k // (4 * UNITS)
            q = k % (4 * UNITS)
            s0 = pl.multiple_of((q // 4) * RU, RU)
            lo = pl.multiple_of((q % 4) * QB, QB)
            return (in_hbm.at[p, pl.ds(s0, RU), pl.ds(lo, QB)],
                    out_hbm.at[p + E - 1, pl.ds(s0, RU), pl.ds(lo, QB)])

        def wait_load(buf, sem):
            pltpu.make_async_copy(x_slice(0), buf, sem).wait()

        def drain_store(buf, sem):
            pltpu.make_async_copy(buf, out_slice(0, 0), sem).wait()

        def rem_load(j, rb):
            src, _ = rem_slices(j)
            pltpu.async_copy(src, rems[rb], sem_rl[rb])

        def rem_step(j, rb):
            wait_load(rems[rb], sem_rl[rb])
            _, dst = rem_slices(j)
            pltpu.async_copy(rems[rb], dst, sem_rs[rb])
            @pl.when(j >= 1)
            def _():
                drain_store(rems[rb ^ 1], sem_rs[rb ^ 1])
            rem_load(j + 1, rb ^ 1)

        def compute4(x_b, o_bufs):
            @plsc.parallel_loop(0, GPB, 1, unroll=GU)
            def _body(g):
                r = g // (QB // L)
                l0 = pl.multiple_of((g % (QB // L)) * L, L)
                x = x_b[r, pl.ds(l0, L)]
                t = x * 255.0 + 8388608.0
                bits = plsc.bitcast(t, jnp.int32)
                for o, dk in zip(o_bufs, dks):
                    o[r, pl.ds(l0, L)] = plsc.load_gather(tab_v, [bits + dk])

        pltpu.async_copy(tab_hbm, tab_v, sem_t)
        pltpu.async_copy(x_slice(0), xs[0], sem_x[0])
        pltpu.async_copy(x_slice(1), xs[1], sem_x[1])
        rem_load(jnp.int32(0), 0)
        pltpu.make_async_copy(tab_hbm, tab_v, sem_t).wait()

        def emb_unit(u, sub):
            wait_load(xs[sub], sem_x[sub])

            @pl.when(u >= 2)
            def _():
                for o in outs[sub]:
                    drain_store(o, sem_o[sub])
            compute4(xs[sub], outs[sub])
            for d_off, o in enumerate(outs[sub]):
                pltpu.async_copy(o, out_slice(d_off, u), sem_o[sub])

            @pl.when(u + 2 < UNITS)
            def _():
                pltpu.async_copy(x_slice(u + 2), xs[sub], sem_x[sub])

        def step(p, carry):
            for sub in (0, 1):
                u = 2 * p + sub
                emb_unit(u, sub)
                for t in range(RPB):
                    rem_step(RPB * u + t, (sub + t) % 2)
            return carry

        lax.fori_loop(0, (UNITS - 1) // 2, step, 0)
        emb_unit(UNITS - 1, 0)
        for t in range(RPB):
            rem_step(RPB * (UNITS - 1) + t, t % 2)
        for j in range(RPB * UNITS, RPB * UNITS + 7):
            rb = j % 2

            @pl.when(j < nrem)
            def _(j=j, rb=rb):
                wait_load(rems[rb], sem_rl[rb])
                _, dst = rem_slices(j)
                pltpu.async_copy(rems[rb], dst, sem_rs[rb])

            @pl.when(j + 1 < nrem)
            def _(j=j, rb=rb):
                drain_store(rems[rb ^ 1], sem_rs[rb ^ 1])
                rem_load(j + 1, rb ^ 1)
        for o in outs[0]:
            drain_store(o, sem_o[0])
        for o in outs[1]:
            drain_store(o, sem_o[1])
        drain_store(rems[0], sem_rs[0])
        drain_store(rems[1], sem_rs[1])

    return sc_fn


_sc_call = _build_sc_call()


def kernel(inputs, table):
    in_pm = inputs.transpose(2, 1, 0)
    tab_dm = table.transpose(1, 0).reshape(E * N_CAT)
    out_pm = _sc_call(in_pm, tab_dm)
    return out_pm.transpose(2, 1, 0)


# speedup vs baseline: 2.3956x; 1.0000x over previous
import functools

import jax
import jax.numpy as jnp
from jax import lax
from jax.experimental import pallas as pl
from jax.experimental.pallas import tpu as pltpu
from jax.experimental.pallas import tpu_sc as plsc

B, S, F = 4096, 200, 27
N_CAT, E = 1000, 32
OUT_F = E + (F - 1)
L = 16
NC, NS = 2, 16
NW = NC * NS
RU = 8
QB = B // 4
UNITS = S // RU
GPB = RU * QB // L
GU = 8
NREM = (F - 1) * UNITS * 4
RPB = 3
EXP_BIAS = 0x4B000000


def _build_sc_call():
    mesh = plsc.VectorSubcoreMesh(core_axis_name="c", subcore_axis_name="s")

    @functools.partial(
        pl.kernel,
        mesh=mesh,
        compiler_params=pltpu.CompilerParams(needs_layout_passes=False),
        out_type=jax.ShapeDtypeStruct((OUT_F, S, B), jnp.float32),
        scratch_types=[
            pltpu.VMEM((E * N_CAT,), jnp.float32),
            [pltpu.VMEM((RU, QB), jnp.float32)] * 2,
            [[pltpu.VMEM((RU, QB), jnp.float32)] * 4] * 2,
            [pltpu.VMEM((RU, QB), jnp.float32)] * 2,
            pltpu.SemaphoreType.DMA,
            [pltpu.SemaphoreType.DMA] * 2,
            [pltpu.SemaphoreType.DMA] * 2,
            [pltpu.SemaphoreType.DMA] * 2,
            [pltpu.SemaphoreType.DMA] * 2,
        ],
    )
    def sc_fn(in_hbm, tab_hbm, out_hbm, tab_v, xs, outs, rems,
              sem_t, sem_x, sem_o, sem_rl, sem_rs):
        wid = lax.axis_index("s") * NC + lax.axis_index("c")
        a4 = (wid // 4) * 4
        lh = (wid % 4) * QB
        dks = [a4 * N_CAT - EXP_BIAS + i * N_CAT for i in range(4)]
        rem0 = wid * NREM // NW
        nrem = (wid + 1) * NREM // NW - rem0

        def x_slice(u):
            return in_hbm.at[0, pl.ds(pl.multiple_of(u * RU, RU), RU),
                             pl.ds(pl.multiple_of(lh, QB), QB)]

        def out_slice(d_off, u):
            return out_hbm.at[a4 + d_off,
                              pl.ds(pl.multiple_of(u * RU, RU), RU),
                              pl.ds(pl.multiple_of(lh, QB), QB)]

        def rem_slices(j):
            k = rem0 + j
            p = 1 + k // (4 * UNITS)
            q = k % (4 * UNITS)
            s0 = pl.multiple_of((q // 4) * RU, RU)
            lo = pl.multiple_of((q % 4) * QB, QB)
            return (in_hbm.at[p, pl.ds(s0, RU), pl.ds(lo, QB)],
                    out_hbm.at[p + E - 1, pl.ds(s0, RU), pl.ds(lo, QB)])

        def wait_load(buf, sem):
            pltpu.make_async_copy(x_slice(0), buf, sem).wait()

        def drain_store(buf, sem):
            pltpu.make_async_copy(buf, out_slice(0, 0), sem).wait()

        def rem_load(j, rb):
            src, _ = rem_slices(j)
            pltpu.async_copy(src, rems[rb], sem_rl[rb])

        def rem_step(j, rb):
            wait_load(rems[rb], sem_rl[rb])
            _, dst = rem_slices(j)
            pltpu.async_copy(rems[rb], dst, sem_rs[rb])
            @pl.when(j >= 1)
            def _():
                drain_store(rems[rb ^ 1], sem_rs[rb ^ 1])
            rem_load(j + 1, rb ^ 1)

        def compute4(x_b, o_bufs):
            @plsc.parallel_loop(0, GPB, 1, unroll=GU)
            def _body(g):
                r = g // (QB // L)
                l0 = pl.multiple_of((g % (QB // L)) * L, L)
                x = x_b[r, pl.ds(l0, L)]
                t = x * 255.0 + 8388608.0
                bits = plsc.bitcast(t, jnp.int32)
                for o, dk in zip(o_bufs, dks):
                    o[r, pl.ds(l0, L)] = plsc.load_gather(tab_v, [bits + dk])

        pltpu.async_copy(tab_hbm, tab_v, sem_t)
        pltpu.async_copy(x_slice(0), xs[0], sem_x[0])
        pltpu.async_copy(x_slice(1), xs[1], sem_x[1])
        rem_load(jnp.int32(0), 0)
        pltpu.make_async_copy(tab_hbm, tab_v, sem_t).wait()

        def emb_unit(u, sub):
            wait_load(xs[sub], sem_x[sub])

            @pl.when(u >= 2)
            def _():
                for o in outs[sub]:
                    drain_store(o, sem_o[sub])
            compute4(xs[sub], outs[sub])
            for d_off, o in enumerate(outs[sub]):
                pltpu.async_copy(o, out_slice(d_off, u), sem_o[sub])

            @pl.when(u + 2 < UNITS)
            def _():
                pltpu.async_copy(x_slice(u + 2), xs[sub], sem_x[sub])

        def step(p, carry):
            for sub in (0, 1):
                u = 2 * p + sub
                emb_unit(u, sub)
                for t in range(RPB):
                    rem_step(RPB * u + t, (sub + t) % 2)
            return carry

        lax.fori_loop(0, (UNITS - 1) // 2, step, 0)
        emb_unit(UNITS - 1, 0)
        for t in range(RPB):
            rem_step(RPB * (UNITS - 1) + t, t % 2)
        for j in range(RPB * UNITS, RPB * UNITS + 7):
            rb = j % 2

            @pl.when(j < nrem)
            def _(j=j, rb=rb):
                wait_load(rems[rb], sem_rl[rb])
                _, dst = rem_slices(j)
                pltpu.async_copy(rems[rb], dst, sem_rs[rb])

            @pl.when(j + 1 < nrem)
            def _(j=j, rb=rb):
                drain_store(rems[rb ^ 1], sem_rs[rb ^ 1])
                rem_load(j + 1, rb ^ 1)
        for o in outs[0]:
            drain_store(o, sem_o[0])
        for o in outs[1]:
            drain_store(o, sem_o[1])
        drain_store(rems[0], sem_rs[0])
        drain_store(rems[1], sem_rs[1])

    return sc_fn


_sc_call = _build_sc_call()


def kernel(inputs, table):
    in_pm = inputs.transpose(2, 1, 0)
    tab_dm = table.transpose(1, 0).reshape(E * N_CAT)
    out_pm = _sc_call(in_pm, tab_dm)
    return out_pm.transpose(2, 1, 0)


# speedup vs baseline: 2.4092x; 1.0057x over previous
import functools

import jax
import jax.numpy as jnp
from jax import lax
from jax.experimental import pallas as pl
from jax.experimental.pallas import tpu as pltpu
from jax.experimental.pallas import tpu_sc as plsc

B, S, F = 4096, 200, 27
N_CAT, E = 1000, 32
OUT_F = E + (F - 1)
L = 16
NC, NS = 2, 16
NW = NC * NS
RU = 8
QB = B // 4
UNITS = S // RU
GPB = RU * QB // L
GU = 4
NREM = (F - 1) * UNITS * 4
RPB = 3
EXP_BIAS = 0x4B000000


def _build_sc_call():
    mesh = plsc.VectorSubcoreMesh(core_axis_name="c", subcore_axis_name="s")

    @functools.partial(
        pl.kernel,
        mesh=mesh,
        compiler_params=pltpu.CompilerParams(needs_layout_passes=False),
        out_type=jax.ShapeDtypeStruct((OUT_F, S, B), jnp.float32),
        scratch_types=[
            pltpu.VMEM((E * N_CAT,), jnp.float32),
            [pltpu.VMEM((RU, QB), jnp.float32)] * 2,
            [[pltpu.VMEM((RU, QB), jnp.float32)] * 4] * 2,
            [pltpu.VMEM((RU, QB), jnp.float32)] * 2,
            pltpu.SemaphoreType.DMA,
            [pltpu.SemaphoreType.DMA] * 2,
            [pltpu.SemaphoreType.DMA] * 2,
            [pltpu.SemaphoreType.DMA] * 2,
            [pltpu.SemaphoreType.DMA] * 2,
        ],
    )
    def sc_fn(in_hbm, tab_hbm, out_hbm, tab_v, xs, outs, rems,
              sem_t, sem_x, sem_o, sem_rl, sem_rs):
        wid = lax.axis_index("s") * NC + lax.axis_index("c")
        a4 = (wid // 4) * 4
        lh = (wid % 4) * QB
        dks = [a4 * N_CAT - EXP_BIAS + i * N_CAT for i in range(4)]
        rem0 = wid * NREM // NW
        nrem = (wid + 1) * NREM // NW - rem0

        def x_slice(u):
            return in_hbm.at[0, pl.ds(pl.multiple_of(u * RU, RU), RU),
                             pl.ds(pl.multiple_of(lh, QB), QB)]

        def out_slice(d_off, u):
            return out_hbm.at[a4 + d_off,
                              pl.ds(pl.multiple_of(u * RU, RU), RU),
                              pl.ds(pl.multiple_of(lh, QB), QB)]

        def rem_slices(j):
            k = rem0 + j
            p = 1 + k // (4 * UNITS)
            q = k % (4 * UNITS)
            s0 = pl.multiple_of((q // 4) * RU, RU)
            lo = pl.multiple_of((q % 4) * QB, QB)
            return (in_hbm.at[p, pl.ds(s0, RU), pl.ds(lo, QB)],
                    out_hbm.at[p + E - 1, pl.ds(s0, RU), pl.ds(lo, QB)])

        def wait_load(buf, sem):
            pltpu.make_async_copy(x_slice(0), buf, sem).wait()

        def drain_store(buf, sem):
            pltpu.make_async_copy(buf, out_slice(0, 0), sem).wait()

        def rem_load(j, rb):
            src, _ = rem_slices(j)
            pltpu.async_copy(src, rems[rb], sem_rl[rb])

        def rem_step(j, rb):
            wait_load(rems[rb], sem_rl[rb])
            _, dst = rem_slices(j)
            pltpu.async_copy(rems[rb], dst, sem_rs[rb])
            @pl.when(j >= 1)
            def _():
                drain_store(rems[rb ^ 1], sem_rs[rb ^ 1])
            rem_load(j + 1, rb ^ 1)

        def compute4(x_b, o_bufs):
            @plsc.parallel_loop(0, GPB, 1, unroll=GU)
            def _body(g):
                r = g // (QB // L)
                l0 = pl.multiple_of((g % (QB // L)) * L, L)
                x = x_b[r, pl.ds(l0, L)]
                t = x * 255.0 + 8388608.0
                bits = plsc.bitcast(t, jnp.int32)
                for o, dk in zip(o_bufs, dks):
                    o[r, pl.ds(l0, L)] = plsc.load_gather(tab_v, [bits + dk])

        pltpu.async_copy(tab_hbm, tab_v, sem_t)
        pltpu.async_copy(x_slice(0), xs[0], sem_x[0])
        pltpu.async_copy(x_slice(1), xs[1], sem_x[1])
        rem_load(jnp.int32(0), 0)
        pltpu.make_async_copy(tab_hbm, tab_v, sem_t).wait()

        def emb_unit(u, sub):
            wait_load(xs[sub], sem_x[sub])

            @pl.when(u >= 2)
            def _():
                for o in outs[sub]:
                    drain_store(o, sem_o[sub])
            compute4(xs[sub], outs[sub])
            for d_off, o in enumerate(outs[sub]):
                pltpu.async_copy(o, out_slice(d_off, u), sem_o[sub])

            @pl.when(u + 2 < UNITS)
            def _():
                pltpu.async_copy(x_slice(u + 2), xs[sub], sem_x[sub])

        def step(p, carry):
            for sub in (0, 1):
                u = 2 * p + sub
                emb_unit(u, sub)
                for t in range(RPB):
                    rem_step(RPB * u + t, (sub + t) % 2)
            return carry

        lax.fori_loop(0, (UNITS - 1) // 2, step, 0)
        emb_unit(UNITS - 1, 0)
        for t in range(RPB):
            rem_step(RPB * (UNITS - 1) + t, t % 2)
        for j in range(RPB * UNITS, RPB * UNITS + 7):
            rb = j % 2

            @pl.when(j < nrem)
            def _(j=j, rb=rb):
                wait_load(rems[rb], sem_rl[rb])
                _, dst = rem_slices(j)
                pltpu.async_copy(rems[rb], dst, sem_rs[rb])

            @pl.when(j + 1 < nrem)
            def _(j=j, rb=rb):
                drain_store(rems[rb ^ 1], sem_rs[rb ^ 1])
                rem_load(j + 1, rb ^ 1)
        for o in outs[0]:
            drain_store(o, sem_o[0])
        for o in outs[1]:
            drain_store(o, sem_o[1])
        drain_store(rems[0], sem_rs[0])
        drain_store(rems[1], sem_rs[1])

    return sc_fn


_sc_call = _build_sc_call()


def kernel(inputs, table):
    in_pm = inputs.transpose(2, 1, 0)
    tab_dm = table.transpose(1, 0).reshape(E * N_CAT)
    out_pm = _sc_call(in_pm, tab_dm)
    return out_pm.transpose(2, 1, 0)
